# baseline (device time: 23992 ns/iter reference)
import jax
import jax.numpy as jnp
from jax import lax
from jax.experimental import pallas as pl
from jax.experimental.pallas import tpu as pltpu

N_DEV = 8
P = 4
IP_HOPS = P - 1
S = 4


def kernel(x, dy):
    m, d_in = x.shape
    _, f = dy.shape
    rows = d_in // N_DEV
    n_streams = 2 * S
    fq = f // n_streams

    stream_dirs = [k % 2 == 0 for k in range(n_streams)]

    def body(x_hbm, dy_hbm, out_ref, acc_ref, x_ref, dy_ref, *rest):
        n = n_streams
        comm = rest[0:n]
        zrecv = rest[n:2 * n]
        ip_send = rest[2 * n:3 * n]
        ip_recv = rest[3 * n:4 * n]
        z_send = rest[4 * n:5 * n]
        z_recv = rest[5 * n:6 * n]
        copy_sems = rest[6 * n]

        xcopy = pltpu.make_async_copy(x_hbm, x_ref, copy_sems.at[n])
        xcopy.start()
        dycopies = []
        for st in range(n_streams):
            c = pltpu.make_async_copy(
                dy_hbm.at[:, pl.ds(st * fq, fq)],
                dy_ref.at[:, pl.ds(st * fq, fq)],
                copy_sems.at[st],
            )
            c.start()
            dycopies.append(c)

        my = lax.axis_index("i")
        r = lax.rem(my, P)
        z = lax.div(my, P)
        left = z * P + lax.rem(r + P - 1, P)
        right = z * P + lax.rem(r + 1, P)
        zpartner = lax.rem(my + P, N_DEV)

        barrier_sem = pltpu.get_barrier_semaphore()
        for nbr in (left, right, zpartner):
            pl.semaphore_signal(
                barrier_sem, inc=1,
                device_id=(nbr,), device_id_type=pl.DeviceIdType.MESH,
            )
        pl.semaphore_wait(barrier_sem, 3)

        def group_at(st, s):
            if stream_dirs[st]:
                return lax.rem(r + P - 1 - s, P)
            return lax.rem(r + 1 + s, P)

        def acc_comp(g, comp_is_mine, st):
            zz = z if comp_is_mine else (1 - z)
            return acc_ref[pl.ds((g + P * zz) * rows, rows),
                           pl.ds(st * fq, fq)]

        def make_ip(st, s):
            return pltpu.make_async_remote_copy(
                src_ref=comm[st].at[s],
                dst_ref=comm[st].at[s + 1],
                send_sem=ip_send[st].at[s],
                recv_sem=ip_recv[st].at[s],
                device_id=(right if stream_dirs[st] else left,),
                device_id_type=pl.DeviceIdType.MESH,
            )

        rdmas = {}
        zdmas = {}

        xcopy.wait()
        for st in range(n_streams):
            dycopies[st].wait()
            acc_ref[:, pl.ds(st * fq, fq)] = lax.dot_general(
                x_ref[:, :].astype(jnp.bfloat16),
                dy_ref[:, pl.ds(st * fq, fq)].astype(jnp.bfloat16),
                dimension_numbers=(((0,), (0,)), ((), ())),
                preferred_element_type=jnp.float32,
            )
            g0 = group_at(st, 0)
            comm[st][0, 0, :, :] = acc_comp(g0, True, st).astype(jnp.bfloat16)
            comm[st][0, 1, :, :] = acc_comp(g0, False, st).astype(jnp.bfloat16)
            rdmas[(st, 0)] = make_ip(st, 0)
            rdmas[(st, 0)].start()

        for s in range(1, IP_HOPS):
            for st in range(n_streams):
                rdmas[(st, s - 1)].wait_recv()
                g = group_at(st, s)
                comm[st][s, 0, :, :] = (
                    comm[st][s, 0, :, :].astype(jnp.float32)
                    + acc_comp(g, True, st)
                ).astype(jnp.bfloat16)
                comm[st][s, 1, :, :] = (
                    comm[st][s, 1, :, :].astype(jnp.float32)
                    + acc_comp(g, False, st)
                ).astype(jnp.bfloat16)
                rdmas[(st, s)] = make_ip(st, s)
                rdmas[(st, s)].start()

        for st in range(n_streams):
            rdmas[(st, IP_HOPS - 1)].wait_recv()
            comm[st][IP_HOPS, 0, :, :] = (
                comm[st][IP_HOPS, 0, :, :].astype(jnp.float32)
                + acc_comp(r, True, st)
            ).astype(jnp.bfloat16)
            comm[st][IP_HOPS, 1, :, :] = (
                comm[st][IP_HOPS, 1, :, :].astype(jnp.float32)
                + acc_comp(r, False, st)
            ).astype(jnp.bfloat16)
            zdmas[st] = pltpu.make_async_remote_copy(
                src_ref=comm[st].at[IP_HOPS, 1],
                dst_ref=zrecv[st],
                send_sem=z_send[st],
                recv_sem=z_recv[st],
                device_id=(zpartner,),
                device_id_type=pl.DeviceIdType.MESH,
            )
            zdmas[st].start()

        for st in range(n_streams):
            zdmas[st].wait_recv()
            out_ref[:, pl.ds(st * fq, fq)] = (
                comm[st][IP_HOPS, 0, :, :].astype(jnp.float32)
                + zrecv[st][:, :].astype(jnp.float32)
            )

        for st in range(n_streams):
            for s in range(IP_HOPS):
                rdmas[(st, s)].wait_send()
            zdmas[st].wait_send()

    return pl.pallas_call(
        body,
        out_shape=jax.ShapeDtypeStruct((rows, f), jnp.float32),
        in_specs=[
            pl.BlockSpec(memory_space=pltpu.MemorySpace.HBM),
            pl.BlockSpec(memory_space=pltpu.MemorySpace.HBM),
        ],
        out_specs=pl.BlockSpec(memory_space=pltpu.VMEM),
        scratch_shapes=(
            [pltpu.VMEM((d_in, f), jnp.float32)]
            + [pltpu.VMEM((m, d_in), jnp.float32)]
            + [pltpu.VMEM((m, f), jnp.float32)]
            + [pltpu.VMEM((IP_HOPS + 1, 2, rows, fq), jnp.bfloat16)
               for _ in range(n_streams)]
            + [pltpu.VMEM((rows, fq), jnp.bfloat16)
               for _ in range(n_streams)]
            + [pltpu.SemaphoreType.DMA((IP_HOPS,))
               for _ in range(n_streams)]
            + [pltpu.SemaphoreType.DMA((IP_HOPS,))
               for _ in range(n_streams)]
            + [pltpu.SemaphoreType.DMA for _ in range(n_streams)]
            + [pltpu.SemaphoreType.DMA for _ in range(n_streams)]
            + [pltpu.SemaphoreType.DMA((n_streams + 1,))]
        ),
        compiler_params=pltpu.CompilerParams(collective_id=0),
    )(x, dy)


# device time: 22424 ns/iter; 1.0699x vs baseline; 1.0699x over previous
import jax
import jax.numpy as jnp
from jax import lax
from jax.experimental import pallas as pl
from jax.experimental.pallas import tpu as pltpu

N_DEV = 8
P = 4
IP_HOPS = P - 1
S = 4


def kernel(x, dy):
    m, d_in = x.shape
    _, f = dy.shape
    rows = d_in // N_DEV
    n_streams = 2 * S
    fq = f // n_streams

    stream_dirs = [k % 2 == 0 for k in range(n_streams)]

    def body(x_ref, dy_ref, out_ref, acc_ref, *rest):
        n = n_streams
        comm = rest[0:n]
        zrecv = rest[n:2 * n]
        ip_send = rest[2 * n:3 * n]
        ip_recv = rest[3 * n:4 * n]
        z_send = rest[4 * n:5 * n]
        z_recv = rest[5 * n:6 * n]

        my = lax.axis_index("i")
        r = lax.rem(my, P)
        z = lax.div(my, P)
        left = z * P + lax.rem(r + P - 1, P)
        right = z * P + lax.rem(r + 1, P)
        zpartner = lax.rem(my + P, N_DEV)

        barrier_sem = pltpu.get_barrier_semaphore()
        for nbr in (left, right, zpartner):
            pl.semaphore_signal(
                barrier_sem, inc=1,
                device_id=(nbr,), device_id_type=pl.DeviceIdType.MESH,
            )

        def group_at(st, s):
            if stream_dirs[st]:
                return lax.rem(r + P - 1 - s, P)
            return lax.rem(r + 1 + s, P)

        def acc_comp(g, comp_is_mine, st):
            zz = z if comp_is_mine else (1 - z)
            return acc_ref[pl.ds((g + P * zz) * rows, rows),
                           pl.ds(st * fq, fq)]

        def make_ip(st, s):
            return pltpu.make_async_remote_copy(
                src_ref=comm[st].at[s],
                dst_ref=comm[st].at[s + 1],
                send_sem=ip_send[st].at[s],
                recv_sem=ip_recv[st].at[s],
                device_id=(right if stream_dirs[st] else left,),
                device_id_type=pl.DeviceIdType.MESH,
            )

        rdmas = {}
        zdmas = {}

        for st in range(n_streams):
            acc_ref[:, pl.ds(st * fq, fq)] = lax.dot_general(
                x_ref[:, :].astype(jnp.bfloat16),
                dy_ref[:, pl.ds(st * fq, fq)].astype(jnp.bfloat16),
                dimension_numbers=(((0,), (0,)), ((), ())),
                preferred_element_type=jnp.float32,
            )
            g0 = group_at(st, 0)
            comm[st][0, 0, :, :] = acc_comp(g0, True, st).astype(jnp.bfloat16)
            comm[st][0, 1, :, :] = acc_comp(g0, False, st).astype(jnp.bfloat16)
            if st == 0:
                pl.semaphore_wait(barrier_sem, 3)
            rdmas[(st, 0)] = make_ip(st, 0)
            rdmas[(st, 0)].start()

        for s in range(1, IP_HOPS):
            for st in range(n_streams):
                rdmas[(st, s - 1)].wait_recv()
                g = group_at(st, s)
                comm[st][s, 0, :, :] = (
                    comm[st][s, 0, :, :].astype(jnp.float32)
                    + acc_comp(g, True, st)
                ).astype(jnp.bfloat16)
                comm[st][s, 1, :, :] = (
                    comm[st][s, 1, :, :].astype(jnp.float32)
                    + acc_comp(g, False, st)
                ).astype(jnp.bfloat16)
                rdmas[(st, s)] = make_ip(st, s)
                rdmas[(st, s)].start()

        for st in range(n_streams):
            rdmas[(st, IP_HOPS - 1)].wait_recv()
            comm[st][IP_HOPS, 0, :, :] = (
                comm[st][IP_HOPS, 0, :, :].astype(jnp.float32)
                + acc_comp(r, True, st)
            ).astype(jnp.bfloat16)
            comm[st][IP_HOPS, 1, :, :] = (
                comm[st][IP_HOPS, 1, :, :].astype(jnp.float32)
                + acc_comp(r, False, st)
            ).astype(jnp.bfloat16)
            zdmas[st] = pltpu.make_async_remote_copy(
                src_ref=comm[st].at[IP_HOPS, 1],
                dst_ref=zrecv[st],
                send_sem=z_send[st],
                recv_sem=z_recv[st],
                device_id=(zpartner,),
                device_id_type=pl.DeviceIdType.MESH,
            )
            zdmas[st].start()

        for st in range(n_streams):
            zdmas[st].wait_recv()
            out_ref[:, pl.ds(st * fq, fq)] = (
                comm[st][IP_HOPS, 0, :, :].astype(jnp.float32)
                + zrecv[st][:, :].astype(jnp.float32)
            )

        for st in range(n_streams):
            for s in range(IP_HOPS):
                rdmas[(st, s)].wait_send()
            zdmas[st].wait_send()

    return pl.pallas_call(
        body,
        out_shape=jax.ShapeDtypeStruct((rows, f), jnp.float32),
        in_specs=[
            pl.BlockSpec(memory_space=pltpu.VMEM),
            pl.BlockSpec(memory_space=pltpu.VMEM),
        ],
        out_specs=pl.BlockSpec(memory_space=pltpu.VMEM),
        scratch_shapes=(
            [pltpu.VMEM((d_in, f), jnp.float32)]
            + [pltpu.VMEM((IP_HOPS + 1, 2, rows, fq), jnp.bfloat16)
               for _ in range(n_streams)]
            + [pltpu.VMEM((rows, fq), jnp.bfloat16)
               for _ in range(n_streams)]
            + [pltpu.SemaphoreType.DMA((IP_HOPS,))
               for _ in range(n_streams)]
            + [pltpu.SemaphoreType.DMA((IP_HOPS,))
               for _ in range(n_streams)]
            + [pltpu.SemaphoreType.DMA for _ in range(n_streams)]
            + [pltpu.SemaphoreType.DMA for _ in range(n_streams)]
        ),
        compiler_params=pltpu.CompilerParams(collective_id=0),
    )(x, dy)
